# baseline (speedup 1.0000x reference)
import jax
import jax.numpy as jnp
from jax import lax
from jax.experimental import pallas as pl
from jax.experimental.pallas import tpu as pltpu
from jax.experimental.pallas import tpu_sc as plsc

VOCAB_SIZE = 1000000
EMBED_DIM = 64
BATCH = 4
SEQ_LEN = 8192

NUM_CORES = 2
NUM_SUBCORES = 16
NUM_WORKERS = NUM_CORES * NUM_SUBCORES
TOTAL = BATCH * SEQ_LEN
B_PER_W = TOTAL // NUM_WORKERS
LANES = 16
EXP3 = 0x01800180


def _body(table_hbm, idx_hbm, out_hbm, idx_v, rows_v, sem):
    wid = lax.axis_index("s") * NUM_CORES + lax.axis_index("c")
    base = wid * B_PER_W
    pltpu.sync_copy(idx_hbm.at[pl.ds(base, B_PER_W)], idx_v)
    pltpu.async_copy(table_hbm.at[idx_v], rows_v, sem).wait()

    scale_vec = jnp.full((LANES,), EXP3, dtype=jnp.int32)

    def scale_row(i, carry):
        for j in range(EMBED_DIM // (2 * LANES)):
            sl = rows_v[i, pl.ds(j * 2 * LANES, 2 * LANES)]
            bits = plsc.bitcast(sl, jnp.int32)
            sl8 = plsc.bitcast(bits + scale_vec, jnp.bfloat16)
            rows_v[i, pl.ds(j * 2 * LANES, 2 * LANES)] = sl8
        return carry

    lax.fori_loop(0, B_PER_W, scale_row, 0)
    pltpu.sync_copy(rows_v, out_hbm.at[pl.ds(base, B_PER_W)])


@jax.jit
def _embed(table_bf, idx):
    mesh = plsc.VectorSubcoreMesh(core_axis_name="c", subcore_axis_name="s")
    run = pl.kernel(
        _body,
        out_type=jax.ShapeDtypeStruct((TOTAL, EMBED_DIM), jnp.bfloat16),
        mesh=mesh,
        scratch_types=[
            pltpu.VMEM((B_PER_W,), jnp.int32),
            pltpu.VMEM((B_PER_W, EMBED_DIM), jnp.bfloat16),
            pltpu.SemaphoreType.DMA,
        ],
        compiler_params=pltpu.CompilerParams(
            use_tc_tiling_on_sc=False, needs_layout_passes=False
        ),
    )
    return run(table_bf, idx)


def kernel(x, input_embedding_table):
    idx = x.reshape(-1).astype(jnp.int32)
    table_bf = input_embedding_table.astype(jnp.bfloat16)
    out = _embed(table_bf, idx)
    return out.astype(jnp.float32).reshape(BATCH, SEQ_LEN, EMBED_DIM)


# speedup vs baseline: 1.2204x; 1.2204x over previous
import jax
import jax.numpy as jnp
from jax import lax
from jax.experimental import pallas as pl
from jax.experimental.pallas import tpu as pltpu
from jax.experimental.pallas import tpu_sc as plsc

VOCAB_SIZE = 1000000
EMBED_DIM = 64
BATCH = 4
SEQ_LEN = 8192
SCALE = 8.0

NUM_CORES = 2
NUM_SUBCORES = 16
NUM_WORKERS = NUM_CORES * NUM_SUBCORES
TOTAL = BATCH * SEQ_LEN
B_PER_W = TOTAL // NUM_WORKERS
LANES = 16
CHUNK = 256
N_CHUNKS = B_PER_W // CHUNK
GROUPS = CHUNK // LANES


def _body(table2_hbm, idx_hbm, out_hbm, idx_v, idx2_v, rows_v, out_v, sem):
    wid = lax.axis_index("s") * NUM_CORES + lax.axis_index("c")
    base = wid * B_PER_W
    lane = lax.iota(jnp.int32, LANES)

    def chunk_body(q, carry):
        cbase = base + q * CHUNK
        pltpu.sync_copy(idx_hbm.at[pl.ds(cbase, CHUNK)], idx_v)

        def halve(k, c2):
            v = idx_v[pl.ds(k * LANES, LANES)]
            idx2_v[pl.ds(k * LANES, LANES)] = v >> 1
            return c2

        lax.fori_loop(0, GROUPS, halve, 0)
        pltpu.async_copy(table2_hbm.at[idx2_v], rows_v, sem).wait()

        def select_group(g, c2):
            i_vec = lane + g * LANES
            par = (idx_v[pl.ds(g * LANES, LANES)] & 1) * EMBED_DIM
            for c in range(EMBED_DIM):
                vals = plsc.load_gather(rows_v, [i_vec, par + c])
                plsc.store_scatter(
                    out_v, [i_vec, jnp.full((LANES,), c, jnp.int32)], vals * SCALE
                )
            return c2

        lax.fori_loop(0, GROUPS, select_group, 0)
        pltpu.sync_copy(out_v, out_hbm.at[pl.ds(cbase, CHUNK)])
        return carry

    lax.fori_loop(0, N_CHUNKS, chunk_body, 0)


@jax.jit
def _embed(table2, idx):
    mesh = plsc.VectorSubcoreMesh(core_axis_name="c", subcore_axis_name="s")
    run = pl.kernel(
        _body,
        out_type=jax.ShapeDtypeStruct((TOTAL, EMBED_DIM), jnp.float32),
        mesh=mesh,
        scratch_types=[
            pltpu.VMEM((CHUNK,), jnp.int32),
            pltpu.VMEM((CHUNK,), jnp.int32),
            pltpu.VMEM((CHUNK, 2 * EMBED_DIM), jnp.float32),
            pltpu.VMEM((CHUNK, EMBED_DIM), jnp.float32),
            pltpu.SemaphoreType.DMA,
        ],
        compiler_params=pltpu.CompilerParams(
            use_tc_tiling_on_sc=True, needs_layout_passes=False
        ),
    )
    return run(table2, idx)


def kernel(x, input_embedding_table):
    idx = x.reshape(-1).astype(jnp.int32)
    table2 = input_embedding_table.reshape(VOCAB_SIZE // 2, 2 * EMBED_DIM)
    out = _embed(table2, idx)
    return out.reshape(BATCH, SEQ_LEN, EMBED_DIM)


# speedup vs baseline: 1.3514x; 1.1074x over previous
import jax
import jax.numpy as jnp
from jax import lax
from jax.experimental import pallas as pl
from jax.experimental.pallas import tpu as pltpu
from jax.experimental.pallas import tpu_sc as plsc

VOCAB_SIZE = 1000000
EMBED_DIM = 64
BATCH = 4
SEQ_LEN = 8192
SCALE = 8.0

NUM_CORES = 2
NUM_SUBCORES = 16
NUM_WORKERS = NUM_CORES * NUM_SUBCORES
TOTAL = BATCH * SEQ_LEN
B_PER_W = TOTAL // NUM_WORKERS
LANES = 16
CHUNK = 64
N_CHUNKS = B_PER_W // CHUNK
GROUPS = CHUNK // LANES
SLABS = VOCAB_SIZE // 8


def _body(table3_hbm, idx_hbm, out_hbm, idx_v, slab_v, rows_v, out_v, sem):
    wid = lax.axis_index("s") * NUM_CORES + lax.axis_index("c")
    base = wid * B_PER_W
    lane = lax.iota(jnp.int32, LANES)

    def chunk_body(q, carry):
        cbase = base + q * CHUNK
        pltpu.sync_copy(idx_hbm.at[pl.ds(cbase, CHUNK)], idx_v)

        def slabify(k, c2):
            v = idx_v[pl.ds(k * LANES, LANES)]
            slab_v[pl.ds(k * LANES, LANES)] = v >> 3
            return c2

        lax.fori_loop(0, GROUPS, slabify, 0)
        pltpu.async_copy(table3_hbm.at[slab_v], rows_v, sem).wait()

        def select_group(g, c2):
            t_vec = lane + g * LANES
            r_vec = idx_v[pl.ds(g * LANES, LANES)] & 7
            for c in range(EMBED_DIM):
                cols = (lane + c) & (EMBED_DIM - 1)
                vals = plsc.load_gather(rows_v, [t_vec, r_vec, cols])
                plsc.store_scatter(out_v, [t_vec, cols], vals * SCALE)
            return c2

        lax.fori_loop(0, GROUPS, select_group, 0)
        pltpu.sync_copy(out_v, out_hbm.at[pl.ds(cbase, CHUNK)])
        return carry

    lax.fori_loop(0, N_CHUNKS, chunk_body, 0)


@jax.jit
def _embed(table3, idx):
    mesh = plsc.VectorSubcoreMesh(core_axis_name="c", subcore_axis_name="s")
    run = pl.kernel(
        _body,
        out_type=jax.ShapeDtypeStruct((TOTAL, EMBED_DIM), jnp.float32),
        mesh=mesh,
        scratch_types=[
            pltpu.VMEM((CHUNK,), jnp.int32),
            pltpu.VMEM((CHUNK,), jnp.int32),
            pltpu.VMEM((CHUNK, 8, 2 * EMBED_DIM), jnp.float32),
            pltpu.VMEM((CHUNK, EMBED_DIM), jnp.float32),
            pltpu.SemaphoreType.DMA,
        ],
        compiler_params=pltpu.CompilerParams(
            use_tc_tiling_on_sc=True, needs_layout_passes=False
        ),
    )
    return run(table3, idx)


def kernel(x, input_embedding_table):
    idx = x.reshape(-1).astype(jnp.int32)
    table3 = jnp.pad(
        input_embedding_table.reshape(SLABS, 8, EMBED_DIM),
        ((0, 0), (0, 0), (0, EMBED_DIM)),
    )
    out = _embed(table3, idx)
    return out.reshape(BATCH, SEQ_LEN, EMBED_DIM)
